# baseline (speedup 1.0000x reference)
import functools

import jax
import jax.numpy as jnp
from jax import lax
from jax.experimental import pallas as pl
from jax.experimental.pallas import tpu as pltpu
from jax.experimental.pallas import tpu_sc as plsc

HIDDEN = 128
BROKEN_DIM = 13
F32 = jnp.float32


def _root_body(fp0_ref, fp1_ref, wr0_ref, wr1_ref, br_ref, wa_ref, wb_ref,
               win_ref, wg0_ref, bin_ref, bg0_ref,
               out_ref, wing0_ref, bing0_ref):
    @pl.when(pl.program_id(0) == 0)
    def _prep():
        wing0_ref[...] = jnp.dot(win_ref[...], wg0_ref[...],
                                 preferred_element_type=F32, precision=jax.lax.Precision.DEFAULT)
        bing0_ref[...] = jnp.dot(bin_ref[...], wg0_ref[...],
                                 preferred_element_type=F32, precision=jax.lax.Precision.DEFAULT) + bg0_ref[...]

    e = jnp.dot(fp0_ref[...], wr0_ref[...], preferred_element_type=F32, precision=jax.lax.Precision.DEFAULT)
    e = e + jnp.dot(fp1_ref[...], wr1_ref[...], preferred_element_type=F32, precision=jax.lax.Precision.DEFAULT)
    e = jnp.maximum(e + br_ref[...], 0.0)
    out_ref[...] = jnp.dot(e, wa_ref[...] + wb_ref[...],
                           preferred_element_type=F32, precision=jax.lax.Precision.DEFAULT)


def _root_term(root_fp, W_root, b_root, W_m0, W_in, W_g0, b_in, b_g0):
    n_roots, fp_dim = root_fp.shape
    rb = n_roots
    for cand in (400, 200, 100, 40, 8):
        if n_roots % cand == 0:
            rb = cand
            break
    grid = n_roots // rb
    h_spec = pl.BlockSpec((HIDDEN, HIDDEN), lambda i: (0, 0))
    v_spec = pl.BlockSpec((1, HIDDEN), lambda i: (0, 0))
    return pl.pallas_call(
        _root_body,
        grid=(grid,),
        in_specs=[
            pl.BlockSpec((rb, fp_dim // 2), lambda i: (i, 0)),
            pl.BlockSpec((rb, fp_dim // 2), lambda i: (i, 1)),
            pl.BlockSpec((fp_dim // 2, HIDDEN), lambda i: (0, 0)),
            pl.BlockSpec((fp_dim // 2, HIDDEN), lambda i: (1, 0)),
            v_spec,
            pl.BlockSpec((HIDDEN, HIDDEN), lambda i: (0, 0)),
            pl.BlockSpec((HIDDEN, HIDDEN), lambda i: (1, 0)),
            h_spec, h_spec, v_spec, v_spec,
        ],
        out_specs=[
            pl.BlockSpec((rb, HIDDEN), lambda i: (i, 0)),
            pl.BlockSpec((HIDDEN, HIDDEN), lambda i: (0, 0)),
            pl.BlockSpec((1, HIDDEN), lambda i: (0, 0)),
        ],
        out_shape=[
            jax.ShapeDtypeStruct((n_roots, HIDDEN), F32),
            jax.ShapeDtypeStruct((HIDDEN, HIDDEN), F32),
            jax.ShapeDtypeStruct((1, HIDDEN), F32),
        ],
    )(root_fp, root_fp, W_root, W_root, b_root.reshape(1, HIDDEN), W_m0, W_m0,
      W_in, W_g0, b_in.reshape(1, HIDDEN), b_g0.reshape(1, HIDDEN))


def _sc_gather(root_term, wdb, ind_pad, brk_pad):
    info = plsc.get_sparse_core_info()
    nc, ns = info.num_cores, info.num_subcores
    nw = nc * ns
    b_pad = ind_pad.shape[0]
    bpw = b_pad // nw

    mesh = plsc.VectorSubcoreMesh(core_axis_name="c", subcore_axis_name="s")

    @functools.partial(
        pl.kernel,
        mesh=mesh,
        out_type=jax.ShapeDtypeStruct((b_pad, HIDDEN), F32),
        scratch_types=[
            pltpu.VMEM((bpw,), jnp.int32),
            pltpu.VMEM((bpw,), jnp.int32),
            pltpu.VMEM((bpw, HIDDEN), F32),
            pltpu.SemaphoreType.DMA,
            pltpu.SemaphoreType.DMA,
            pltpu.SemaphoreType.DMA,
        ],
    )
    def k(rt_hbm, wdb_hbm, ind_hbm, brk_hbm, eb_hbm,
          idx_v, brk_v, rows_v, sem_a, sem_b, sem_c):
        wid = lax.axis_index("s") * nc + lax.axis_index("c")
        base = wid * bpw
        ci = pltpu.async_copy(ind_hbm.at[pl.ds(base, bpw)], idx_v, sem_c)
        ck = pltpu.async_copy(brk_hbm.at[pl.ds(base, bpw)], brk_v, sem_b)
        ci.wait()
        ca = pltpu.async_copy(rt_hbm.at[idx_v], rows_v, sem_a)
        ck.wait()
        ca.wait()
        cb = pltpu.async_copy(wdb_hbm.at[brk_v], rows_v, sem_b, add=True)
        cb.wait()
        pltpu.async_copy(rows_v, eb_hbm.at[pl.ds(base, bpw)], sem_c).wait()

    return k(root_term, wdb, ind_pad, brk_pad)


BF16 = jnp.bfloat16


def _node_body(x_ref, eb_ref, scale_ref,
               wg0_ref, bg0_ref, wg1_ref, bg1_ref,
               wb_ref, wc_ref, wm1_ref, bm1_ref, wo_ref, bo_ref, out_ref,
               pool_ref, rep_ref, mask_ref):
    fb, na = out_ref.shape
    rows = fb * na

    @pl.when(pl.program_id(0) == 0)
    def _init():
        frag_row = lax.broadcasted_iota(jnp.int32, (fb, rows), 1) // na
        fid = lax.broadcasted_iota(jnp.int32, (fb, rows), 0)
        pool_ref[...] = (frag_row == fid).astype(F32)
        frag_col = lax.broadcasted_iota(jnp.int32, (rows, fb), 0) // na
        fid2 = lax.broadcasted_iota(jnp.int32, (rows, fb), 1)
        rep_ref[...] = (frag_col == fid2).astype(F32)
        atom_row = lax.broadcasted_iota(jnp.int32, (rows, na), 0) % na
        aid = lax.broadcasted_iota(jnp.int32, (rows, na), 1)
        mask_ref[...] = (atom_row == aid).astype(F32)

    t = jnp.dot(x_ref[...], wg0_ref[...], preferred_element_type=F32, precision=jax.lax.Precision.DEFAULT) + bg0_ref[...]
    t = jnp.maximum(t, 0.0)
    t = jnp.dot(t, wg1_ref[...], preferred_element_type=F32, precision=jax.lax.Precision.DEFAULT) + bg1_ref[...]
    t = jnp.maximum(t, 0.0)

    hc = jnp.dot(t, wc_ref[...], preferred_element_type=F32, precision=jax.lax.Precision.DEFAULT)

    avg = jnp.dot(pool_ref[...], t, preferred_element_type=F32, precision=jax.lax.Precision.DEFAULT) * scale_ref[0, 0]

    r0 = eb_ref[...] - jnp.dot(
        avg, wb_ref[...], preferred_element_type=F32, precision=jax.lax.Precision.DEFAULT)

    rep = jnp.dot(rep_ref[...], r0, preferred_element_type=F32, precision=jax.lax.Precision.DEFAULT)

    h = jnp.maximum(hc + rep, 0.0)
    h = jnp.dot(h, wm1_ref[...], preferred_element_type=F32, precision=jax.lax.Precision.DEFAULT) + bm1_ref[...]
    h = jnp.maximum(h, 0.0)
    red = jnp.sum(h * wo_ref[...], axis=1, keepdims=True) + bo_ref[0, 0]
    g2 = mask_ref[...] * red
    out2 = jnp.dot(pool_ref[...], g2, preferred_element_type=F32, precision=jax.lax.Precision.DEFAULT)
    out_ref[...] = 1.0 / (1.0 + jnp.exp(-out2))


def _node_pipeline(node_h, eb_term, scale, b,
                   W_ing0, b_ing0, W_g1, b_g1,
                   W_m0, W_m1, b_m1, wo_col, bo_11):
    n = node_h.shape[0]
    na = n // b
    fb = b
    for cand in (40, 8):
        if b % cand == 0 and (cand * na) % 8 == 0:
            fb = cand
            break
    rows = fb * na
    grid = b // fb

    h128 = HIDDEN
    w_spec = pl.BlockSpec((h128, h128), lambda i: (0, 0))
    b_spec = pl.BlockSpec((1, h128), lambda i: (0, 0))
    return pl.pallas_call(
        _node_body,
        grid=(grid,),
        in_specs=[
            pl.BlockSpec((rows, h128), lambda i: (i, 0)),
            pl.BlockSpec((fb, h128), lambda i: (i, 0)),
            pl.BlockSpec((1, 1), lambda i: (0, 0)),
            w_spec, b_spec, w_spec, b_spec,
            pl.BlockSpec((h128, h128), lambda i: (1, 0)),
            pl.BlockSpec((h128, h128), lambda i: (2, 0)),
            w_spec, b_spec, b_spec,
            pl.BlockSpec((1, 1), lambda i: (0, 0)),
        ],
        out_specs=pl.BlockSpec((fb, na), lambda i: (i, 0)),
        out_shape=jax.ShapeDtypeStruct((b, na), F32),
        scratch_shapes=[
            pltpu.VMEM((fb, rows), F32),
            pltpu.VMEM((rows, fb), F32),
            pltpu.VMEM((rows, na), F32),
        ],
    )(node_h, eb_term, scale,
      W_ing0, b_ing0,
      W_g1, b_g1.reshape(1, h128), W_m0, W_m0, W_m1,
      b_m1.reshape(1, h128), wo_col, bo_11)


def kernel(node_h, root_fp, ind_maps, broken, n_atoms,
           W_root, b_root, W_in, b_in, W_g0, b_g0, W_g1, b_g1,
           W_m0, b_m0, W_m1, b_m1, W_o, b_o):
    b = ind_maps.shape[0]
    na = node_h.shape[0] // b

    Wdb = W_m0[3 * HIDDEN:] + b_m0[None, :]

    root_term, W_ing0, b_ing0 = _root_term(
        root_fp, W_root, b_root, W_m0, W_in, W_g0, b_in, b_g0)

    b_pad = -(-b // 256) * 256
    ind_pad = jnp.pad(ind_maps.astype(jnp.int32), (0, b_pad - b))
    brk_pad = jnp.pad(broken.astype(jnp.int32), (0, b_pad - b))
    eb_term = _sc_gather(root_term, Wdb, ind_pad, brk_pad)

    scale = (1.0 / jnp.asarray(n_atoms).astype(F32)).reshape(1, 1)
    bo_11 = b_o.reshape(1, 1).astype(F32)

    return _node_pipeline(node_h, eb_term, scale, b,
                          W_ing0, b_ing0, W_g1, b_g1,
                          W_m0, W_m1, wo_col=W_o.reshape(1, HIDDEN),
                          bo_11=bo_11, b_m1=b_m1)


# speedup vs baseline: 1.0316x; 1.0316x over previous
import functools

import jax
import jax.numpy as jnp
from jax import lax
from jax.experimental import pallas as pl
from jax.experimental.pallas import tpu as pltpu
from jax.experimental.pallas import tpu_sc as plsc

HIDDEN = 128
BROKEN_DIM = 13
F32 = jnp.float32


def _root_body(fp0_ref, fp1_ref, wr0_ref, wr1_ref, br_ref, wa_ref, wb_ref,
               win_ref, wg0_ref, bin_ref, bg0_ref,
               out_ref, wing0_ref, bing0_ref):
    @pl.when(pl.program_id(0) == 0)
    def _prep():
        wing0_ref[...] = jnp.dot(win_ref[...], wg0_ref[...],
                                 preferred_element_type=F32, precision=jax.lax.Precision.DEFAULT)
        bing0_ref[...] = jnp.dot(bin_ref[...], wg0_ref[...],
                                 preferred_element_type=F32, precision=jax.lax.Precision.DEFAULT) + bg0_ref[...]

    e = jnp.dot(fp0_ref[...], wr0_ref[...], preferred_element_type=F32, precision=jax.lax.Precision.DEFAULT)
    e = e + jnp.dot(fp1_ref[...], wr1_ref[...], preferred_element_type=F32, precision=jax.lax.Precision.DEFAULT)
    e = jnp.maximum(e + br_ref[...], 0.0)
    out_ref[...] = jnp.dot(e, wa_ref[...] + wb_ref[...],
                           preferred_element_type=F32, precision=jax.lax.Precision.DEFAULT)


def _root_term(root_fp, W_root, b_root, W_m0, W_in, W_g0, b_in, b_g0):
    n_roots, fp_dim = root_fp.shape
    rb = n_roots
    for cand in (400, 200, 100, 40, 8):
        if n_roots % cand == 0:
            rb = cand
            break
    grid = n_roots // rb
    h_spec = pl.BlockSpec((HIDDEN, HIDDEN), lambda i: (0, 0))
    v_spec = pl.BlockSpec((1, HIDDEN), lambda i: (0, 0))
    return pl.pallas_call(
        _root_body,
        grid=(grid,),
        in_specs=[
            pl.BlockSpec((rb, fp_dim // 2), lambda i: (i, 0)),
            pl.BlockSpec((rb, fp_dim // 2), lambda i: (i, 1)),
            pl.BlockSpec((fp_dim // 2, HIDDEN), lambda i: (0, 0)),
            pl.BlockSpec((fp_dim // 2, HIDDEN), lambda i: (1, 0)),
            v_spec,
            pl.BlockSpec((HIDDEN, HIDDEN), lambda i: (0, 0)),
            pl.BlockSpec((HIDDEN, HIDDEN), lambda i: (1, 0)),
            h_spec, h_spec, v_spec, v_spec,
        ],
        out_specs=[
            pl.BlockSpec((rb, HIDDEN), lambda i: (i, 0)),
            pl.BlockSpec((HIDDEN, HIDDEN), lambda i: (0, 0)),
            pl.BlockSpec((1, HIDDEN), lambda i: (0, 0)),
        ],
        out_shape=[
            jax.ShapeDtypeStruct((n_roots, HIDDEN), F32),
            jax.ShapeDtypeStruct((HIDDEN, HIDDEN), F32),
            jax.ShapeDtypeStruct((1, HIDDEN), F32),
        ],
    )(root_fp, root_fp, W_root, W_root, b_root.reshape(1, HIDDEN), W_m0, W_m0,
      W_in, W_g0, b_in.reshape(1, HIDDEN), b_g0.reshape(1, HIDDEN))


def _sc_bpw(b, nw):
    bpw = -(-b // nw)
    while bpw <= b:
        if bpw % 8 == 0 and b % bpw == 0:
            return bpw
        bpw += 1
    return b


def _sc_gather(root_term, wdb, ind, brk):
    info = plsc.get_sparse_core_info()
    nc, ns = info.num_cores, info.num_subcores
    nw = nc * ns
    b = ind.shape[0]
    bpw = _sc_bpw(b, nw)
    n_active = b // bpw

    mesh = plsc.VectorSubcoreMesh(core_axis_name="c", subcore_axis_name="s")

    @functools.partial(
        pl.kernel,
        mesh=mesh,
        out_type=jax.ShapeDtypeStruct((b, HIDDEN), F32),
        scratch_types=[
            pltpu.VMEM((bpw,), jnp.int32),
            pltpu.VMEM((bpw,), jnp.int32),
            pltpu.VMEM((bpw, HIDDEN), F32),
            pltpu.SemaphoreType.DMA,
            pltpu.SemaphoreType.DMA,
            pltpu.SemaphoreType.DMA,
        ],
    )
    def k(rt_hbm, wdb_hbm, ind_hbm, brk_hbm, eb_hbm,
          idx_v, brk_v, rows_v, sem_a, sem_b, sem_c):
        wid = lax.axis_index("s") * nc + lax.axis_index("c")

        @pl.when(wid < n_active)
        def _work():
            base = wid * bpw
            ci = pltpu.async_copy(ind_hbm.at[pl.ds(base, bpw)], idx_v, sem_c)
            ck = pltpu.async_copy(brk_hbm.at[pl.ds(base, bpw)], brk_v, sem_b)
            ci.wait()
            ca = pltpu.async_copy(rt_hbm.at[idx_v], rows_v, sem_a)
            ck.wait()
            ca.wait()
            cb = pltpu.async_copy(wdb_hbm.at[brk_v], rows_v, sem_b, add=True)
            cb.wait()
            pltpu.async_copy(rows_v, eb_hbm.at[pl.ds(base, bpw)], sem_c).wait()

    return k(root_term, wdb, ind, brk)


BF16 = jnp.bfloat16


def _node_body(x_ref, eb_ref, scale_ref,
               wg0_ref, bg0_ref, wg1_ref, bg1_ref,
               wb_ref, wc_ref, wm1_ref, bm1_ref, wo_ref, bo_ref, out_ref,
               pool_ref, rep_ref, mask_ref):
    fb, na = out_ref.shape
    rows = fb * na

    @pl.when(pl.program_id(0) == 0)
    def _init():
        frag_row = lax.broadcasted_iota(jnp.int32, (fb, rows), 1) // na
        fid = lax.broadcasted_iota(jnp.int32, (fb, rows), 0)
        pool_ref[...] = (frag_row == fid).astype(F32)
        frag_col = lax.broadcasted_iota(jnp.int32, (rows, fb), 0) // na
        fid2 = lax.broadcasted_iota(jnp.int32, (rows, fb), 1)
        rep_ref[...] = (frag_col == fid2).astype(F32)
        atom_row = lax.broadcasted_iota(jnp.int32, (rows, na), 0) % na
        aid = lax.broadcasted_iota(jnp.int32, (rows, na), 1)
        mask_ref[...] = (atom_row == aid).astype(F32)

    t = jnp.dot(x_ref[...], wg0_ref[...], preferred_element_type=F32, precision=jax.lax.Precision.DEFAULT) + bg0_ref[...]
    t = jnp.maximum(t, 0.0)
    t = jnp.dot(t, wg1_ref[...], preferred_element_type=F32, precision=jax.lax.Precision.DEFAULT) + bg1_ref[...]
    t = jnp.maximum(t, 0.0)

    hc = jnp.dot(t, wc_ref[...], preferred_element_type=F32, precision=jax.lax.Precision.DEFAULT)

    avg = jnp.dot(pool_ref[...], t, preferred_element_type=F32, precision=jax.lax.Precision.DEFAULT) * scale_ref[0, 0]

    r0 = eb_ref[...] - jnp.dot(
        avg, wb_ref[...], preferred_element_type=F32, precision=jax.lax.Precision.DEFAULT)

    rep = jnp.dot(rep_ref[...], r0, preferred_element_type=F32, precision=jax.lax.Precision.DEFAULT)

    h = jnp.maximum(hc + rep, 0.0)
    h = jnp.dot(h, wm1_ref[...], preferred_element_type=F32, precision=jax.lax.Precision.DEFAULT) + bm1_ref[...]
    h = jnp.maximum(h, 0.0)
    red = jnp.sum(h * wo_ref[...], axis=1, keepdims=True) + bo_ref[0, 0]
    g2 = mask_ref[...] * red
    out2 = jnp.dot(pool_ref[...], g2, preferred_element_type=F32, precision=jax.lax.Precision.DEFAULT)
    out_ref[...] = 1.0 / (1.0 + jnp.exp(-out2))


def _node_pipeline(node_h, eb_term, scale, b,
                   W_ing0, b_ing0, W_g1, b_g1,
                   W_m0, W_m1, b_m1, wo_col, bo_11):
    n = node_h.shape[0]
    na = n // b
    fb = b
    for cand in (40, 8):
        if b % cand == 0 and (cand * na) % 8 == 0:
            fb = cand
            break
    rows = fb * na
    grid = b // fb

    h128 = HIDDEN
    w_spec = pl.BlockSpec((h128, h128), lambda i: (0, 0))
    b_spec = pl.BlockSpec((1, h128), lambda i: (0, 0))
    return pl.pallas_call(
        _node_body,
        grid=(grid,),
        in_specs=[
            pl.BlockSpec((rows, h128), lambda i: (i, 0)),
            pl.BlockSpec((fb, h128), lambda i: (i, 0)),
            pl.BlockSpec((1, 1), lambda i: (0, 0)),
            w_spec, b_spec, w_spec, b_spec,
            pl.BlockSpec((h128, h128), lambda i: (1, 0)),
            pl.BlockSpec((h128, h128), lambda i: (2, 0)),
            w_spec, b_spec, b_spec,
            pl.BlockSpec((1, 1), lambda i: (0, 0)),
        ],
        out_specs=pl.BlockSpec((fb, na), lambda i: (i, 0)),
        out_shape=jax.ShapeDtypeStruct((b, na), F32),
        scratch_shapes=[
            pltpu.VMEM((fb, rows), F32),
            pltpu.VMEM((rows, fb), F32),
            pltpu.VMEM((rows, na), F32),
        ],
    )(node_h, eb_term, scale,
      W_ing0, b_ing0,
      W_g1, b_g1.reshape(1, h128), W_m0, W_m0, W_m1,
      b_m1.reshape(1, h128), wo_col, bo_11)


def kernel(node_h, root_fp, ind_maps, broken, n_atoms,
           W_root, b_root, W_in, b_in, W_g0, b_g0, W_g1, b_g1,
           W_m0, b_m0, W_m1, b_m1, W_o, b_o):
    b = ind_maps.shape[0]
    na = node_h.shape[0] // b

    Wdb = W_m0[3 * HIDDEN:] + b_m0[None, :]

    root_term, W_ing0, b_ing0 = _root_term(
        root_fp, W_root, b_root, W_m0, W_in, W_g0, b_in, b_g0)

    eb_term = _sc_gather(root_term, Wdb,
                         ind_maps.astype(jnp.int32), broken.astype(jnp.int32))

    scale = (1.0 / jnp.asarray(n_atoms).astype(F32)).reshape(1, 1)
    bo_11 = b_o.reshape(1, 1).astype(F32)

    return _node_pipeline(node_h, eb_term, scale, b,
                          W_ing0, b_ing0, W_g1, b_g1,
                          W_m0, W_m1, wo_col=W_o.reshape(1, HIDDEN),
                          bo_11=bo_11, b_m1=b_m1)


# speedup vs baseline: 1.0392x; 1.0073x over previous
import functools

import jax
import jax.numpy as jnp
from jax import lax
from jax.experimental import pallas as pl
from jax.experimental.pallas import tpu as pltpu
from jax.experimental.pallas import tpu_sc as plsc

HIDDEN = 128
BROKEN_DIM = 13
F32 = jnp.float32


def _root_body(fp0_ref, fp1_ref, wr0_ref, wr1_ref, br_ref, wa_ref, wb_ref,
               wm0t_ref, bm0_ref, win_ref, wg0_ref, bin_ref, bg0_ref,
               out_ref, wing0_ref, bing0_ref, wdb_ref):
    @pl.when(pl.program_id(0) == 0)
    def _prep():
        wing0_ref[...] = jnp.dot(win_ref[...], wg0_ref[...],
                                 preferred_element_type=F32, precision=jax.lax.Precision.DEFAULT)
        bing0_ref[...] = jnp.dot(bin_ref[...], wg0_ref[...],
                                 preferred_element_type=F32, precision=jax.lax.Precision.DEFAULT) + bg0_ref[...]
        wdb_ref[...] = wm0t_ref[...][:wdb_ref.shape[0]] + bm0_ref[...]

    e = jnp.dot(fp0_ref[...], wr0_ref[...], preferred_element_type=F32, precision=jax.lax.Precision.DEFAULT)
    e = e + jnp.dot(fp1_ref[...], wr1_ref[...], preferred_element_type=F32, precision=jax.lax.Precision.DEFAULT)
    e = jnp.maximum(e + br_ref[...], 0.0)
    out_ref[...] = jnp.dot(e, wa_ref[...] + wb_ref[...],
                           preferred_element_type=F32, precision=jax.lax.Precision.DEFAULT)


def _root_term(root_fp, W_root, b_root, W_m0, b_m0, W_in, W_g0, b_in, b_g0):
    n_roots, fp_dim = root_fp.shape
    rb = n_roots
    for cand in (400, 200, 100, 40, 8):
        if n_roots % cand == 0:
            rb = cand
            break
    grid = n_roots // rb
    h_spec = pl.BlockSpec((HIDDEN, HIDDEN), lambda i: (0, 0))
    v_spec = pl.BlockSpec((1, HIDDEN), lambda i: (0, 0))
    return pl.pallas_call(
        _root_body,
        grid=(grid,),
        in_specs=[
            pl.BlockSpec((rb, fp_dim // 2), lambda i: (i, 0)),
            pl.BlockSpec((rb, fp_dim // 2), lambda i: (i, 1)),
            pl.BlockSpec((fp_dim // 2, HIDDEN), lambda i: (0, 0)),
            pl.BlockSpec((fp_dim // 2, HIDDEN), lambda i: (1, 0)),
            v_spec,
            pl.BlockSpec((HIDDEN, HIDDEN), lambda i: (0, 0)),
            pl.BlockSpec((HIDDEN, HIDDEN), lambda i: (1, 0)),
            pl.BlockSpec((HIDDEN, HIDDEN), lambda i: (3, 0)),
            v_spec,
            h_spec, h_spec, v_spec, v_spec,
        ],
        out_specs=[
            pl.BlockSpec((rb, HIDDEN), lambda i: (i, 0)),
            pl.BlockSpec((HIDDEN, HIDDEN), lambda i: (0, 0)),
            pl.BlockSpec((1, HIDDEN), lambda i: (0, 0)),
            pl.BlockSpec((BROKEN_DIM, HIDDEN), lambda i: (0, 0)),
        ],
        out_shape=[
            jax.ShapeDtypeStruct((n_roots, HIDDEN), F32),
            jax.ShapeDtypeStruct((HIDDEN, HIDDEN), F32),
            jax.ShapeDtypeStruct((1, HIDDEN), F32),
            jax.ShapeDtypeStruct((BROKEN_DIM, HIDDEN), F32),
        ],
    )(root_fp, root_fp, W_root, W_root, b_root.reshape(1, HIDDEN), W_m0, W_m0,
      W_m0, b_m0.reshape(1, HIDDEN),
      W_in, W_g0, b_in.reshape(1, HIDDEN), b_g0.reshape(1, HIDDEN))


def _sc_bpw(b, nw):
    bpw = -(-b // nw)
    while bpw <= b:
        if bpw % 8 == 0 and b % bpw == 0:
            return bpw
        bpw += 1
    return b


def _sc_gather(root_term, wdb, ind, brk):
    info = plsc.get_sparse_core_info()
    nc, ns = info.num_cores, info.num_subcores
    nw = nc * ns
    b = ind.shape[0]
    bpw = _sc_bpw(b, nw)
    n_active = b // bpw

    mesh = plsc.VectorSubcoreMesh(core_axis_name="c", subcore_axis_name="s")

    @functools.partial(
        pl.kernel,
        mesh=mesh,
        out_type=jax.ShapeDtypeStruct((b, HIDDEN), F32),
        scratch_types=[
            pltpu.VMEM((bpw,), jnp.int32),
            pltpu.VMEM((bpw,), jnp.int32),
            pltpu.VMEM((bpw, HIDDEN), F32),
            pltpu.SemaphoreType.DMA,
            pltpu.SemaphoreType.DMA,
            pltpu.SemaphoreType.DMA,
        ],
    )
    def k(rt_hbm, wdb_hbm, ind_hbm, brk_hbm, eb_hbm,
          idx_v, brk_v, rows_v, sem_a, sem_b, sem_c):
        wid = lax.axis_index("s") * nc + lax.axis_index("c")

        @pl.when(wid < n_active)
        def _work():
            base = wid * bpw
            ci = pltpu.async_copy(ind_hbm.at[pl.ds(base, bpw)], idx_v, sem_c)
            ck = pltpu.async_copy(brk_hbm.at[pl.ds(base, bpw)], brk_v, sem_b)
            ci.wait()
            ca = pltpu.async_copy(rt_hbm.at[idx_v], rows_v, sem_a)
            ck.wait()
            ca.wait()
            cb = pltpu.async_copy(wdb_hbm.at[brk_v], rows_v, sem_b, add=True)
            cb.wait()
            pltpu.async_copy(rows_v, eb_hbm.at[pl.ds(base, bpw)], sem_c).wait()

    return k(root_term, wdb, ind, brk)


def _node_body(x_ref, eb_ref, scale_ref,
               wg0_ref, bg0_ref, wg1_ref, bg1_ref,
               wb_ref, wc_ref, wm1_ref, bm1_ref, wo_ref, bo_ref, out_ref,
               pool_ref, rep_ref, mask_ref):
    fb, na = out_ref.shape
    rows = fb * na

    @pl.when(pl.program_id(0) == 0)
    def _init():
        frag_row = lax.broadcasted_iota(jnp.int32, (fb, rows), 1) // na
        fid = lax.broadcasted_iota(jnp.int32, (fb, rows), 0)
        pool_ref[...] = (frag_row == fid).astype(F32)
        frag_col = lax.broadcasted_iota(jnp.int32, (rows, fb), 0) // na
        fid2 = lax.broadcasted_iota(jnp.int32, (rows, fb), 1)
        rep_ref[...] = (frag_col == fid2).astype(F32)
        atom_row = lax.broadcasted_iota(jnp.int32, (rows, na), 0) % na
        aid = lax.broadcasted_iota(jnp.int32, (rows, na), 1)
        mask_ref[...] = (atom_row == aid).astype(F32)

    t = jnp.dot(x_ref[...], wg0_ref[...], preferred_element_type=F32, precision=jax.lax.Precision.DEFAULT) + bg0_ref[...]
    t = jnp.maximum(t, 0.0)
    t = jnp.dot(t, wg1_ref[...], preferred_element_type=F32, precision=jax.lax.Precision.DEFAULT) + bg1_ref[...]
    t = jnp.maximum(t, 0.0)

    hc = jnp.dot(t, wc_ref[...], preferred_element_type=F32, precision=jax.lax.Precision.DEFAULT)

    avg = jnp.dot(pool_ref[...], t, preferred_element_type=F32, precision=jax.lax.Precision.DEFAULT) * scale_ref[0, 0]

    r0 = eb_ref[...] - jnp.dot(
        avg, wb_ref[...], preferred_element_type=F32, precision=jax.lax.Precision.DEFAULT)

    rep = jnp.dot(rep_ref[...], r0, preferred_element_type=F32, precision=jax.lax.Precision.DEFAULT)

    h = jnp.maximum(hc + rep, 0.0)
    h = jnp.dot(h, wm1_ref[...], preferred_element_type=F32, precision=jax.lax.Precision.DEFAULT) + bm1_ref[...]
    h = jnp.maximum(h, 0.0)
    red = jnp.sum(h * wo_ref[...], axis=1, keepdims=True) + bo_ref[0, 0]
    g2 = mask_ref[...] * red
    out2 = jnp.dot(pool_ref[...], g2, preferred_element_type=F32, precision=jax.lax.Precision.DEFAULT)
    out_ref[...] = 1.0 / (1.0 + jnp.exp(-out2))


def _node_pipeline(node_h, eb_term, scale, b,
                   W_ing0, b_ing0, W_g1, b_g1,
                   W_m0, W_m1, b_m1, wo_col, bo_11):
    n = node_h.shape[0]
    na = n // b
    fb = b
    for cand in (40, 8):
        if b % cand == 0 and (cand * na) % 8 == 0:
            fb = cand
            break
    rows = fb * na
    grid = b // fb

    h128 = HIDDEN
    w_spec = pl.BlockSpec((h128, h128), lambda i: (0, 0))
    b_spec = pl.BlockSpec((1, h128), lambda i: (0, 0))
    return pl.pallas_call(
        _node_body,
        grid=(grid,),
        in_specs=[
            pl.BlockSpec((rows, h128), lambda i: (i, 0)),
            pl.BlockSpec((fb, h128), lambda i: (i, 0)),
            pl.BlockSpec((1, 1), lambda i: (0, 0)),
            w_spec, b_spec, w_spec, b_spec,
            pl.BlockSpec((h128, h128), lambda i: (1, 0)),
            pl.BlockSpec((h128, h128), lambda i: (2, 0)),
            w_spec, b_spec, b_spec,
            pl.BlockSpec((1, 1), lambda i: (0, 0)),
        ],
        out_specs=pl.BlockSpec((fb, na), lambda i: (i, 0)),
        out_shape=jax.ShapeDtypeStruct((b, na), F32),
        scratch_shapes=[
            pltpu.VMEM((fb, rows), F32),
            pltpu.VMEM((rows, fb), F32),
            pltpu.VMEM((rows, na), F32),
        ],
    )(node_h, eb_term, scale,
      W_ing0, b_ing0,
      W_g1, b_g1.reshape(1, h128), W_m0, W_m0, W_m1,
      b_m1.reshape(1, h128), wo_col, bo_11)


def kernel(node_h, root_fp, ind_maps, broken, n_atoms,
           W_root, b_root, W_in, b_in, W_g0, b_g0, W_g1, b_g1,
           W_m0, b_m0, W_m1, b_m1, W_o, b_o):
    b = ind_maps.shape[0]
    na = node_h.shape[0] // b

    root_term, W_ing0, b_ing0, Wdb = _root_term(
        root_fp, W_root, b_root, W_m0, b_m0, W_in, W_g0, b_in, b_g0)

    eb_term = _sc_gather(root_term, Wdb,
                         ind_maps.astype(jnp.int32), broken.astype(jnp.int32))

    scale = (1.0 / jnp.asarray(n_atoms).astype(F32)).reshape(1, 1)
    bo_11 = b_o.reshape(1, 1).astype(F32)

    return _node_pipeline(node_h, eb_term, scale, b,
                          W_ing0, b_ing0, W_g1, b_g1,
                          W_m0, W_m1, wo_col=W_o.reshape(1, HIDDEN),
                          bo_11=bo_11, b_m1=b_m1)
